# baseline (speedup 1.0000x reference)
import functools

import jax
import jax.numpy as jnp
from jax import lax
from jax.experimental import pallas as pl
from jax.experimental.pallas import tpu as pltpu
from jax.experimental.pallas import tpu_sc as plsc

N = 50000
D = 128
N_PAD = 50688
NCHUNK = 8
CHUNK = 6336
SPM_ROWS = CHUNK + 16
STRIPE = CHUNK // 16
E = 800000
BLK = 1792
E_PAD = 802816
PER_TILE = E_PAD // 16
N_BLOCKS = PER_TILE // BLK
DEG_PER_TILE = E_PAD // 32
R_BLK = 768


def _sc_mesh():
    return plsc.VectorSubcoreMesh(core_axis_name="c", subcore_axis_name="s")


_SC_PARAMS = pltpu.CompilerParams(needs_layout_passes=False)


def _deg_body(dst_hbm, hist_hbm, hist_v, eb_dst):
    c = lax.axis_index("c")
    s = lax.axis_index("s")
    wid = s * 2 + c
    zero16 = jnp.zeros((16,), jnp.float32)

    def zh(i, _):
        hist_v[pl.ds(i * 16, 16)] = zero16
        return 0
    lax.fori_loop(0, N_PAD // 16, zh, 0)

    one16 = jnp.ones((16,), jnp.float32)

    def blk(b, _):
        e0 = wid * DEG_PER_TILE + b * BLK
        pltpu.sync_copy(dst_hbm.at[pl.ds(e0, BLK)], eb_dst)

        def grp(g, _2):
            dv = eb_dst[pl.ds(g * 16, 16)]
            plsc.addupdate_scatter(hist_v, [dv], one16)
            return 0
        lax.fori_loop(0, BLK // 16, grp, 0)
        return 0
    lax.fori_loop(0, 14, blk, 0)
    pltpu.sync_copy(hist_v, hist_hbm.at[wid])


def _sc_deg(dstp):
    f = functools.partial(
        pl.kernel,
        out_type=jax.ShapeDtypeStruct((32, N_PAD), jnp.float32),
        mesh=_sc_mesh(),
        scratch_types=[
            pltpu.VMEM((N_PAD,), jnp.float32),
            pltpu.VMEM((BLK,), jnp.int32),
        ],
        compiler_params=_SC_PARAMS,
    )
    return f(_deg_body)(dstp)


CAP = PER_TILE // 128 + 2


def _bin_body(src_hbm, dst_hbm, gbin, sbin, cnt,
              eb_sa, eb_da, eb_sb, eb_db, rg, rs, cbuf, esem, fsem):
    c = lax.axis_index("c")
    s = lax.axis_index("s")
    i16 = lax.iota(jnp.int32, 16)
    zi16 = jnp.zeros((16,), jnp.int32)
    for r in range(8):
        for j in range(8):
            cbuf[r, pl.ds(j * 16, 16)] = zi16
    ebase = s * PER_TILE
    padg = i16 + s * 16
    pads = i16 + jnp.int32(CHUNK)
    chunk_u = jnp.uint32(CHUNK)

    def start_edges(b, es, ed):
        e0 = ebase + b * BLK
        pltpu.async_copy(src_hbm.at[pl.ds(e0, BLK)], es, esem)
        pltpu.async_copy(dst_hbm.at[pl.ds(e0, BLK)], ed, esem)

    def wait_edges():
        for _ in range(2):
            pltpu.make_async_copy(src_hbm.at[pl.ds(0, BLK)], eb_sa,
                                  esem).wait()

    def wait_flush():
        pltpu.make_async_copy(rg.at[0, pl.ds(0, 1)],
                              gbin.at[0, 0, 0, pl.ds(0, 1)], fsem).wait()

    def process(b, es, ed, nxt_es, nxt_ed, carry):
        wait_edges()

        @pl.when(b + 1 < N_BLOCKS)
        def _prefetch():
            start_edges(b + 1, nxt_es, nxt_ed)

        def grp(g, cy):
            w0, w1, w2, w3, o = cy
            ws = [w0, w1, w2, w3]
            sv = es[pl.ds(g * 16, 16)]
            dv = ed[pl.ds(g * 16, 16)]
            for k in range(4):
                w = ws[k]
                dl = dv - (4 * c + k) * CHUNK
                m = plsc.bitcast(dl, jnp.uint32) < chunk_u
                cum = plsc.cumsum(m.astype(jnp.int32))
                p = w + cum - 1
                rowv = lax.shift_right_logical(p, 7) & 3
                colv = p & 127
                plsc.store_scatter(rg.at[k], [rowv, colv], sv, mask=m)
                plsc.store_scatter(rs.at[k], [rowv, colv], dl, mask=m)
                wn = w + cum[15]
                r0 = lax.shift_right_logical(w, 7)
                crossed = lax.shift_right_logical(wn, 7) > r0
                full = crossed & (o >= 6)

                @pl.when(full)
                def _drain():
                    wait_flush()
                    wait_flush()

                @pl.when(crossed)
                def _flush():
                    pltpu.async_copy(rg.at[k, pl.ds(r0 & 3, 1)],
                                     gbin.at[c, s, k, pl.ds(r0, 1)], fsem)
                    pltpu.async_copy(rs.at[k, pl.ds(r0 & 3, 1)],
                                     sbin.at[c, s, k, pl.ds(r0, 1)], fsem)
                o = o + 2 * crossed.astype(jnp.int32) \
                    - 2 * full.astype(jnp.int32)
                ws[k] = wn
            return (ws[0], ws[1], ws[2], ws[3], o)
        return lax.fori_loop(0, BLK // 16, grp, carry)

    start_edges(0, eb_sa, eb_da)

    def super_body(t, cy):
        cy = process(2 * t, eb_sa, eb_da, eb_sb, eb_db, cy)
        cy = process(2 * t + 1, eb_sb, eb_db, eb_sa, eb_da, cy)
        return cy
    w0, w1, w2, w3, o = lax.fori_loop(0, N_BLOCKS // 2, super_body,
                                      (0, 0, 0, 0, 0))

    ws = [w0, w1, w2, w3]
    for k in range(4):
        w = ws[k]
        r0 = lax.shift_right_logical(w, 7)
        end = (r0 + 1) * 128
        for j in range(8):
            p = w + j * 16 + i16
            m = p < end
            rowv = lax.shift_right_logical(p, 7) & 3
            colv = p & 127
            plsc.store_scatter(rg.at[k], [rowv, colv], padg, mask=m)
            plsc.store_scatter(rs.at[k], [rowv, colv], pads, mask=m)

        @pl.when((w & 127) > 0)
        def _flush_tail():
            pltpu.async_copy(rg.at[k, pl.ds(r0 & 3, 1)],
                             gbin.at[c, s, k, pl.ds(r0, 1)], fsem)
            pltpu.async_copy(rs.at[k, pl.ds(r0 & 3, 1)],
                             sbin.at[c, s, k, pl.ds(r0, 1)], fsem)
        o = o + 2 * ((w & 127) > 0).astype(jnp.int32)

        nf = lax.shift_right_logical(w + 127, 7)
        nfv = jnp.zeros((16,), jnp.int32) + nf
        for j in range(8):
            cbuf[0, pl.ds(j * 16, 16)] = nfv
        pltpu.sync_copy(cbuf, cnt.at[c, s, k])

    def drain(i, _):
        wait_flush()
        return 0
    lax.fori_loop(0, o, drain, 0)


def _sc_bin(srcp, dstp):
    f = functools.partial(
        pl.kernel,
        out_type=(
            jax.ShapeDtypeStruct((2, 16, 4, CAP, 128), jnp.int32),
            jax.ShapeDtypeStruct((2, 16, 4, CAP, 128), jnp.int32),
            jax.ShapeDtypeStruct((2, 16, 4, 8, 128), jnp.int32),
        ),
        mesh=_sc_mesh(),
        scratch_types=[
            pltpu.VMEM((BLK,), jnp.int32),
            pltpu.VMEM((BLK,), jnp.int32),
            pltpu.VMEM((BLK,), jnp.int32),
            pltpu.VMEM((BLK,), jnp.int32),
            pltpu.VMEM((4, 4, 128), jnp.int32),
            pltpu.VMEM((4, 4, 128), jnp.int32),
            pltpu.VMEM((8, 128), jnp.int32),
            pltpu.SemaphoreType.DMA,
            pltpu.SemaphoreType.DMA,
        ],
        compiler_params=_SC_PARAMS,
    )
    return f(_bin_body)(srcp, dstp)


def _apply_body(y_hbm, gbin, sbin, cnt, acc_hbm,
                gidxv, sidxv, cbuf, rows, spm, isem, gsem, ssem, zsem):
    c = lax.axis_index("c")
    s = lax.axis_index("s")
    zero16 = jnp.zeros((16,), jnp.float32)

    def start_idx(ci, j, slot):
        pltpu.async_copy(gbin.at[c, s, ci, pl.ds(j, 1)],
                         gidxv.at[pl.ds(slot, 1)], isem)
        pltpu.async_copy(sbin.at[c, s, ci, pl.ds(j, 1)],
                         sidxv.at[pl.ds(slot, 1)], isem)

    def wait_idx():
        for _ in range(2):
            pltpu.make_async_copy(gbin.at[0, 0, 0, pl.ds(0, 1)],
                                  gidxv.at[pl.ds(0, 1)], isem).wait()

    def wait_gather():
        pltpu.make_async_copy(y_hbm.at[gidxv.at[0]], rows.at[0],
                              gsem).wait()

    def wait_scatter():
        pltpu.make_async_copy(rows.at[0], spm.at[sidxv.at[0]], ssem).wait()

    for ci in range(NCHUNK // 2):
        chunk = (NCHUNK // 2) * c + ci
        lo = chunk * CHUNK

        def zz(r, _):
            for j in range(D // 16):
                rows[0, r, pl.ds(j * 16, 16)] = zero16
            return 0
        lax.fori_loop(0, 128, zz, 0)
        z0 = s * 528

        @pl.when(s < 12)
        def _zero():
            for k in range(4):
                pltpu.async_copy(rows.at[0],
                                 spm.at[pl.ds(z0 + k * 128, 128)], zsem)
            pltpu.async_copy(rows.at[0, pl.ds(0, 16)],
                             spm.at[pl.ds(z0 + 512, 16)], zsem)
            for k in range(4):
                pltpu.make_async_copy(rows.at[0], spm.at[pl.ds(0, 128)],
                                      zsem).wait()
            pltpu.make_async_copy(rows.at[0, pl.ds(0, 16)],
                                  spm.at[pl.ds(0, 16)], zsem).wait()
        plsc.subcore_barrier()

        pltpu.sync_copy(cnt.at[c, s, ci], cbuf)
        nf = cbuf[0, pl.ds(0, 16)][0]

        @pl.when(nf > 0)
        def _run():
            start_idx(ci, 0, 0)

            @pl.when(nf > 1)
            def _i1():
                start_idx(ci, 1, 1)
            wait_idx()
            pltpu.async_copy(y_hbm.at[gidxv.at[0]], rows.at[0], gsem)

            @pl.when(nf > 1)
            def _g1():
                wait_idx()
                pltpu.async_copy(y_hbm.at[gidxv.at[1]], rows.at[1], gsem)

            @pl.when(nf > 2)
            def _i2():
                start_idx(ci, 2, 2)

            def fire(j, _2):
                wait_gather()

                @pl.when(j >= 1)
                def _ws():
                    wait_scatter()

                @pl.when(j + 2 < nf)
                def _g2():
                    wait_idx()
                    pltpu.async_copy(y_hbm.at[gidxv.at[(j + 2) & 3]],
                                     rows.at[(j + 2) & 3], gsem)

                @pl.when(j + 3 < nf)
                def _i3():
                    start_idx(ci, j + 3, (j + 3) & 3)
                pltpu.async_copy(rows.at[j & 3], spm.at[sidxv.at[j & 3]],
                                 ssem, add=True)
                return 0
            lax.fori_loop(0, nf, fire, 0)
            wait_scatter()
        plsc.subcore_barrier()

        cb = lo + z0

        @pl.when(s < 12)
        def _out():
            for k in range(4):
                pltpu.async_copy(spm.at[pl.ds(z0 + k * 128, 128)],
                                 acc_hbm.at[pl.ds(cb + k * 128, 128)], zsem)
            pltpu.async_copy(spm.at[pl.ds(z0 + 512, 16)],
                             acc_hbm.at[pl.ds(cb + 512, 16)], zsem)
            for k in range(4):
                pltpu.make_async_copy(spm.at[pl.ds(0, 128)],
                                      acc_hbm.at[pl.ds(0, 128)], zsem).wait()
            pltpu.make_async_copy(spm.at[pl.ds(0, 16)],
                                  acc_hbm.at[pl.ds(0, 16)], zsem).wait()
        plsc.subcore_barrier()


def _sc_apply(y, gbin, sbin, cnt):
    f = functools.partial(
        pl.kernel,
        out_type=jax.ShapeDtypeStruct((N_PAD, D), jnp.float32),
        mesh=_sc_mesh(),
        scratch_types=[
            pltpu.VMEM((4, 128), jnp.int32),
            pltpu.VMEM((4, 128), jnp.int32),
            pltpu.VMEM((8, 128), jnp.int32),
            pltpu.VMEM((4, 128, D), jnp.float32),
            pltpu.VMEM_SHARED((SPM_ROWS, D), jnp.float32),
            pltpu.SemaphoreType.DMA,
            pltpu.SemaphoreType.DMA,
            pltpu.SemaphoreType.DMA,
            pltpu.SemaphoreType.DMA,
        ],
        compiler_params=_SC_PARAMS,
    )
    return f(_apply_body)(y, gbin, sbin, cnt)


def _k1_body(hist_ref, x_ref, w_ref, y_ref, dinv_ref):
    ones = jnp.ones((32, 1), jnp.float32)
    deg = lax.dot_general(hist_ref[...], ones, (((0,), (0,)), ((), ())),
                          preferred_element_type=jnp.float32)
    dinv = lax.rsqrt(deg + 1.0)
    dinv_ref[...] = dinv
    xw = jnp.dot(x_ref[...], w_ref[...], preferred_element_type=jnp.float32)
    y_ref[...] = xw * dinv


def _tc_k1(hist, xp, W1):
    return pl.pallas_call(
        _k1_body,
        grid=(N_PAD // R_BLK,),
        in_specs=[
            pl.BlockSpec((32, R_BLK), lambda j: (0, j)),
            pl.BlockSpec((R_BLK, D), lambda j: (j, 0)),
            pl.BlockSpec((D, D), lambda j: (0, 0)),
        ],
        out_specs=[
            pl.BlockSpec((R_BLK, D), lambda j: (j, 0)),
            pl.BlockSpec((R_BLK, 1), lambda j: (j, 0)),
        ],
        out_shape=(
            jax.ShapeDtypeStruct((N_PAD, D), jnp.float32),
            jax.ShapeDtypeStruct((N_PAD, 1), jnp.float32),
        ),
    )(hist, xp, W1)


def _mid_body(acc_ref, y_ref, d_ref, b_ref, w_ref, o_ref):
    d = d_ref[...]
    h = jnp.maximum(d * (acc_ref[...] + y_ref[...]) + b_ref[...], 0.0)
    o_ref[...] = jnp.dot(h, w_ref[...], preferred_element_type=jnp.float32) * d


def _tc_mid(acc, y, dinv, b, W):
    return pl.pallas_call(
        _mid_body,
        grid=(N_PAD // R_BLK,),
        in_specs=[
            pl.BlockSpec((R_BLK, D), lambda j: (j, 0)),
            pl.BlockSpec((R_BLK, D), lambda j: (j, 0)),
            pl.BlockSpec((R_BLK, 1), lambda j: (j, 0)),
            pl.BlockSpec((1, D), lambda j: (0, 0)),
            pl.BlockSpec((D, D), lambda j: (0, 0)),
        ],
        out_specs=pl.BlockSpec((R_BLK, D), lambda j: (j, 0)),
        out_shape=jax.ShapeDtypeStruct((N_PAD, D), jnp.float32),
    )(acc, y, dinv, b.reshape(1, D), W)


def _post_body(acc_ref, y_ref, d_ref, b_ref, wv1_ref, bv1_ref, wv2_ref,
               bv2_ref, wa1_ref, ba1_ref, wa2_ref, ba2_ref, q_ref, gsum):
    j = pl.program_id(0)
    h = jnp.maximum(d_ref[...] * (acc_ref[...] + y_ref[...]) + b_ref[...], 0.0)
    rows = lax.broadcasted_iota(jnp.int32, (R_BLK, 1), 0) + j * R_BLK
    h = jnp.where(rows < N, h, 0.0)
    part = jnp.sum(h, axis=0, keepdims=True)

    @pl.when(j == 0)
    def _init():
        gsum[...] = part
        q_ref[...] = jnp.zeros((1, D), jnp.float32)

    @pl.when(j > 0)
    def _acc():
        gsum[...] = gsum[...] + part

    @pl.when(j == N_PAD // R_BLK - 1)
    def _final():
        g = gsum[...] * (1.0 / N)
        hv = jnp.maximum(
            jnp.dot(g, wv1_ref[...], preferred_element_type=jnp.float32)
            + bv1_ref[...], 0.0)
        v = jnp.dot(hv, wv2_ref[...], preferred_element_type=jnp.float32) \
            + bv2_ref[...]
        ha = jnp.maximum(
            jnp.dot(g, wa1_ref[...], preferred_element_type=jnp.float32)
            + ba1_ref[...], 0.0)
        a = jnp.dot(ha, wa2_ref[...], preferred_element_type=jnp.float32) \
            + ba2_ref[...]
        cols = lax.broadcasted_iota(jnp.int32, (1, D), 1)
        amean = jnp.sum(jnp.where(cols < 6, a, 0.0)) * (1.0 / 6.0)
        q_ref[...] = v[:, 0:1] + a - amean


def _tc_post(acc, y, dinv, b3, Wv1, bv1, Wv2p, bv2p, Wa1, ba1, Wa2p, ba2p):
    whole = pl.BlockSpec((1, D), lambda j: (0, 0))
    mat = pl.BlockSpec((D, D), lambda j: (0, 0))
    return pl.pallas_call(
        _post_body,
        grid=(N_PAD // R_BLK,),
        in_specs=[
            pl.BlockSpec((R_BLK, D), lambda j: (j, 0)),
            pl.BlockSpec((R_BLK, D), lambda j: (j, 0)),
            pl.BlockSpec((R_BLK, 1), lambda j: (j, 0)),
            whole, mat, whole, mat, whole, mat, whole, mat, whole,
        ],
        out_specs=pl.BlockSpec((1, D), lambda j: (0, 0)),
        out_shape=jax.ShapeDtypeStruct((1, D), jnp.float32),
        scratch_shapes=[pltpu.VMEM((1, D), jnp.float32)],
    )(acc, y, dinv, b3.reshape(1, D), Wv1, bv1.reshape(1, D), Wv2p,
      bv2p, Wa1, ba1.reshape(1, D), Wa2p, ba2p)


def kernel(x, edge_index, W1, b1, W2, b2, W3, b3,
           Wv1, bv1, Wv2, bv2, Wa1, ba1, Wa2, ba2):
    src = edge_index[0].astype(jnp.int32)
    dst = edge_index[1].astype(jnp.int32)
    n_extra = E_PAD - E
    pad_src = (jnp.arange(n_extra, dtype=jnp.int32) * 37) % 1024
    pad_dst = jnp.full((n_extra,), N_PAD - 1, jnp.int32)
    srcp = jnp.concatenate([src, pad_src])
    dstp = jnp.concatenate([dst, pad_dst])
    xp = jnp.pad(x, ((0, N_PAD - N), (0, 0)))

    bv2p = jnp.pad(bv2, (0, D - 1)).reshape(1, D)
    Wv2p = jnp.pad(Wv2, ((0, 0), (0, D - 1)))
    ba2p = jnp.pad(ba2, (0, D - 6)).reshape(1, D)
    Wa2p = jnp.pad(Wa2, ((0, 0), (0, D - 6)))

    hist = _sc_deg(dstp)
    gbin, sbin, cnt = _sc_bin(srcp, dstp)
    y1, dinv = _tc_k1(hist, xp, W1)
    acc1 = _sc_apply(y1, gbin, sbin, cnt)
    y2 = _tc_mid(acc1, y1, dinv, b1, W2)
    acc2 = _sc_apply(y2, gbin, sbin, cnt)
    y3 = _tc_mid(acc2, y2, dinv, b2, W3)
    acc3 = _sc_apply(y3, gbin, sbin, cnt)
    q = _tc_post(acc3, y3, dinv, b3, Wv1, bv1, Wv2p, bv2p, Wa1, ba1,
                 Wa2p, ba2p)
    return q[:, :6]


# speedup vs baseline: 1.0511x; 1.0511x over previous
import functools

import jax
import jax.numpy as jnp
from jax import lax
from jax.experimental import pallas as pl
from jax.experimental.pallas import tpu as pltpu
from jax.experimental.pallas import tpu_sc as plsc

N = 50000
D = 128
N_PAD = 50688
NCHUNK = 6
CHUNK = 8448
SPM_ROWS = CHUNK + 16
STRIPE = CHUNK // 16
E = 800000
BLK = 1792
E_PAD = 802816
PER_TILE = E_PAD // 16
N_BLOCKS = PER_TILE // BLK
DEG_PER_TILE = E_PAD // 32
R_BLK = 768


def _sc_mesh():
    return plsc.VectorSubcoreMesh(core_axis_name="c", subcore_axis_name="s")


_SC_PARAMS = pltpu.CompilerParams(needs_layout_passes=False)


def _deg_body(dst_hbm, hist_hbm, hist_v, eb_dst):
    c = lax.axis_index("c")
    s = lax.axis_index("s")
    wid = s * 2 + c
    zero16 = jnp.zeros((16,), jnp.float32)

    def zh(i, _):
        hist_v[pl.ds(i * 16, 16)] = zero16
        return 0
    lax.fori_loop(0, N_PAD // 16, zh, 0)

    one16 = jnp.ones((16,), jnp.float32)

    def blk(b, _):
        e0 = wid * DEG_PER_TILE + b * BLK
        pltpu.sync_copy(dst_hbm.at[pl.ds(e0, BLK)], eb_dst)

        def grp(g, _2):
            dv = eb_dst[pl.ds(g * 16, 16)]
            plsc.addupdate_scatter(hist_v, [dv], one16)
            return 0
        lax.fori_loop(0, BLK // 16, grp, 0)
        return 0
    lax.fori_loop(0, 14, blk, 0)
    pltpu.sync_copy(hist_v, hist_hbm.at[wid])


def _sc_deg(dstp):
    f = functools.partial(
        pl.kernel,
        out_type=jax.ShapeDtypeStruct((32, N_PAD), jnp.float32),
        mesh=_sc_mesh(),
        scratch_types=[
            pltpu.VMEM((N_PAD,), jnp.float32),
            pltpu.VMEM((BLK,), jnp.int32),
        ],
        compiler_params=_SC_PARAMS,
    )
    return f(_deg_body)(dstp)


CAP = PER_TILE // 128 + 2


def _bin_body(src_hbm, dst_hbm, gbin, sbin, cnt,
              eb_sa, eb_da, eb_sb, eb_db, rg, rs, cbuf, esem, fsem):
    c = lax.axis_index("c")
    s = lax.axis_index("s")
    i16 = lax.iota(jnp.int32, 16)
    zi16 = jnp.zeros((16,), jnp.int32)
    for r in range(8):
        for j in range(8):
            cbuf[r, pl.ds(j * 16, 16)] = zi16
    ebase = s * PER_TILE
    padg = i16 + s * 16
    pads = i16 + jnp.int32(CHUNK)
    chunk_u = jnp.uint32(CHUNK)

    def start_edges(b, es, ed):
        e0 = ebase + b * BLK
        pltpu.async_copy(src_hbm.at[pl.ds(e0, BLK)], es, esem)
        pltpu.async_copy(dst_hbm.at[pl.ds(e0, BLK)], ed, esem)

    def wait_edges():
        for _ in range(2):
            pltpu.make_async_copy(src_hbm.at[pl.ds(0, BLK)], eb_sa,
                                  esem).wait()

    def wait_flush():
        pltpu.make_async_copy(rg.at[0, pl.ds(0, 1)],
                              gbin.at[0, 0, 0, pl.ds(0, 1)], fsem).wait()

    def process(b, es, ed, nxt_es, nxt_ed, carry):
        wait_edges()

        @pl.when(b + 1 < N_BLOCKS)
        def _prefetch():
            start_edges(b + 1, nxt_es, nxt_ed)

        def grp(g, cy):
            w0, w1, w2, o = cy
            ws = [w0, w1, w2]
            sv = es[pl.ds(g * 16, 16)]
            dv = ed[pl.ds(g * 16, 16)]
            for k in range(3):
                w = ws[k]
                dl = dv - (3 * c + k) * CHUNK
                m = plsc.bitcast(dl, jnp.uint32) < chunk_u
                cum = plsc.cumsum(m.astype(jnp.int32))
                p = w + cum - 1
                rowv = lax.shift_right_logical(p, 7) & 3
                colv = p & 127
                plsc.store_scatter(rg.at[k], [rowv, colv], sv, mask=m)
                plsc.store_scatter(rs.at[k], [rowv, colv], dl, mask=m)
                wn = w + cum[15]
                r0 = lax.shift_right_logical(w, 7)
                crossed = lax.shift_right_logical(wn, 7) > r0
                full = crossed & (o >= 6)

                @pl.when(full)
                def _drain():
                    wait_flush()
                    wait_flush()

                @pl.when(crossed)
                def _flush():
                    pltpu.async_copy(rg.at[k, pl.ds(r0 & 3, 1)],
                                     gbin.at[c, s, k, pl.ds(r0, 1)], fsem)
                    pltpu.async_copy(rs.at[k, pl.ds(r0 & 3, 1)],
                                     sbin.at[c, s, k, pl.ds(r0, 1)], fsem)
                o = o + 2 * crossed.astype(jnp.int32) \
                    - 2 * full.astype(jnp.int32)
                ws[k] = wn
            return (ws[0], ws[1], ws[2], o)
        return lax.fori_loop(0, BLK // 16, grp, carry)

    start_edges(0, eb_sa, eb_da)

    def super_body(t, cy):
        cy = process(2 * t, eb_sa, eb_da, eb_sb, eb_db, cy)
        cy = process(2 * t + 1, eb_sb, eb_db, eb_sa, eb_da, cy)
        return cy
    w0, w1, w2, o = lax.fori_loop(0, N_BLOCKS // 2, super_body,
                                  (0, 0, 0, 0))

    ws = [w0, w1, w2]
    for k in range(3):
        w = ws[k]
        r0 = lax.shift_right_logical(w, 7)
        end = (r0 + 1) * 128
        for j in range(8):
            p = w + j * 16 + i16
            m = p < end
            rowv = lax.shift_right_logical(p, 7) & 3
            colv = p & 127
            plsc.store_scatter(rg.at[k], [rowv, colv], padg, mask=m)
            plsc.store_scatter(rs.at[k], [rowv, colv], pads, mask=m)

        @pl.when((w & 127) > 0)
        def _flush_tail():
            pltpu.async_copy(rg.at[k, pl.ds(r0 & 3, 1)],
                             gbin.at[c, s, k, pl.ds(r0, 1)], fsem)
            pltpu.async_copy(rs.at[k, pl.ds(r0 & 3, 1)],
                             sbin.at[c, s, k, pl.ds(r0, 1)], fsem)
        o = o + 2 * ((w & 127) > 0).astype(jnp.int32)

        nf = lax.shift_right_logical(w + 127, 7)
        nfv = jnp.zeros((16,), jnp.int32) + nf
        for j in range(8):
            cbuf[0, pl.ds(j * 16, 16)] = nfv
        pltpu.sync_copy(cbuf, cnt.at[c, s, k])

    def drain(i, _):
        wait_flush()
        return 0
    lax.fori_loop(0, o, drain, 0)


def _sc_bin(srcp, dstp):
    f = functools.partial(
        pl.kernel,
        out_type=(
            jax.ShapeDtypeStruct((2, 16, 3, CAP, 128), jnp.int32),
            jax.ShapeDtypeStruct((2, 16, 3, CAP, 128), jnp.int32),
            jax.ShapeDtypeStruct((2, 16, 3, 8, 128), jnp.int32),
        ),
        mesh=_sc_mesh(),
        scratch_types=[
            pltpu.VMEM((BLK,), jnp.int32),
            pltpu.VMEM((BLK,), jnp.int32),
            pltpu.VMEM((BLK,), jnp.int32),
            pltpu.VMEM((BLK,), jnp.int32),
            pltpu.VMEM((3, 4, 128), jnp.int32),
            pltpu.VMEM((3, 4, 128), jnp.int32),
            pltpu.VMEM((8, 128), jnp.int32),
            pltpu.SemaphoreType.DMA,
            pltpu.SemaphoreType.DMA,
        ],
        compiler_params=_SC_PARAMS,
    )
    return f(_bin_body)(srcp, dstp)


def _apply_body(y_hbm, gbin, sbin, cnt, acc_hbm,
                gidxv, sidxv, cbuf, rows, spm, isem, gsem, ssem, zsem):
    c = lax.axis_index("c")
    s = lax.axis_index("s")
    zero16 = jnp.zeros((16,), jnp.float32)

    def start_idx(ci, j, slot):
        pltpu.async_copy(gbin.at[c, s, ci, pl.ds(j, 1)],
                         gidxv.at[pl.ds(slot, 1)], isem)
        pltpu.async_copy(sbin.at[c, s, ci, pl.ds(j, 1)],
                         sidxv.at[pl.ds(slot, 1)], isem)

    def wait_idx():
        for _ in range(2):
            pltpu.make_async_copy(gbin.at[0, 0, 0, pl.ds(0, 1)],
                                  gidxv.at[pl.ds(0, 1)], isem).wait()

    def wait_gather():
        pltpu.make_async_copy(y_hbm.at[gidxv.at[0]], rows.at[0],
                              gsem).wait()

    def wait_scatter():
        pltpu.make_async_copy(rows.at[0], spm.at[sidxv.at[0]], ssem).wait()

    for ci in range(NCHUNK // 2):
        chunk = (NCHUNK // 2) * c + ci
        lo = chunk * CHUNK

        def zz(r, _):
            for j in range(D // 16):
                rows[0, r, pl.ds(j * 16, 16)] = zero16
            return 0
        lax.fori_loop(0, 128, zz, 0)
        z0 = s * STRIPE
        for k in range(4):
            pltpu.async_copy(rows.at[0], spm.at[pl.ds(z0 + k * 128, 128)],
                             zsem)
        pltpu.async_copy(rows.at[0, pl.ds(0, 16)],
                         spm.at[pl.ds(z0 + 512, 16)], zsem)
        for k in range(4):
            pltpu.make_async_copy(rows.at[0], spm.at[pl.ds(0, 128)],
                                  zsem).wait()
        pltpu.make_async_copy(rows.at[0, pl.ds(0, 16)],
                              spm.at[pl.ds(0, 16)], zsem).wait()
        plsc.subcore_barrier()

        pltpu.sync_copy(cnt.at[c, s, ci], cbuf)
        nf = cbuf[0, pl.ds(0, 16)][0]

        @pl.when(nf > 0)
        def _run():
            start_idx(ci, 0, 0)

            @pl.when(nf > 1)
            def _i1():
                start_idx(ci, 1, 1)
            wait_idx()
            pltpu.async_copy(y_hbm.at[gidxv.at[0]], rows.at[0], gsem)

            @pl.when(nf > 1)
            def _g1():
                wait_idx()
                pltpu.async_copy(y_hbm.at[gidxv.at[1]], rows.at[1], gsem)

            @pl.when(nf > 2)
            def _i2():
                start_idx(ci, 2, 2)

            def fire(j, _2):
                jm = lax.rem(j, 3)
                wait_gather()

                @pl.when(j >= 1)
                def _ws():
                    wait_scatter()

                @pl.when(j + 2 < nf)
                def _g2():
                    wait_idx()
                    pltpu.async_copy(y_hbm.at[gidxv.at[(j + 2) & 3]],
                                     rows.at[lax.rem(j + 2, 3)], gsem)

                @pl.when(j + 3 < nf)
                def _i3():
                    start_idx(ci, j + 3, (j + 3) & 3)
                pltpu.async_copy(rows.at[jm], spm.at[sidxv.at[j & 3]],
                                 ssem, add=True)
                return 0
            lax.fori_loop(0, nf, fire, 0)
            wait_scatter()
        plsc.subcore_barrier()

        cb = lo + z0
        for k in range(4):
            pltpu.async_copy(spm.at[pl.ds(z0 + k * 128, 128)],
                             acc_hbm.at[pl.ds(cb + k * 128, 128)], zsem)
        pltpu.async_copy(spm.at[pl.ds(z0 + 512, 16)],
                         acc_hbm.at[pl.ds(cb + 512, 16)], zsem)
        for k in range(4):
            pltpu.make_async_copy(spm.at[pl.ds(0, 128)],
                                  acc_hbm.at[pl.ds(0, 128)], zsem).wait()
        pltpu.make_async_copy(spm.at[pl.ds(0, 16)],
                              acc_hbm.at[pl.ds(0, 16)], zsem).wait()
        plsc.subcore_barrier()


def _sc_apply(y, gbin, sbin, cnt):
    f = functools.partial(
        pl.kernel,
        out_type=jax.ShapeDtypeStruct((N_PAD, D), jnp.float32),
        mesh=_sc_mesh(),
        scratch_types=[
            pltpu.VMEM((4, 128), jnp.int32),
            pltpu.VMEM((4, 128), jnp.int32),
            pltpu.VMEM((8, 128), jnp.int32),
            pltpu.VMEM((3, 128, D), jnp.float32),
            pltpu.VMEM_SHARED((SPM_ROWS, D), jnp.float32),
            pltpu.SemaphoreType.DMA,
            pltpu.SemaphoreType.DMA,
            pltpu.SemaphoreType.DMA,
            pltpu.SemaphoreType.DMA,
        ],
        compiler_params=_SC_PARAMS,
    )
    return f(_apply_body)(y, gbin, sbin, cnt)


def _k1_body(hist_ref, x_ref, w_ref, y_ref, dinv_ref):
    ones = jnp.ones((32, 1), jnp.float32)
    deg = lax.dot_general(hist_ref[...], ones, (((0,), (0,)), ((), ())),
                          preferred_element_type=jnp.float32)
    dinv = lax.rsqrt(deg + 1.0)
    dinv_ref[...] = dinv
    xw = jnp.dot(x_ref[...], w_ref[...], preferred_element_type=jnp.float32)
    y_ref[...] = xw * dinv


def _tc_k1(hist, xp, W1):
    return pl.pallas_call(
        _k1_body,
        grid=(N_PAD // R_BLK,),
        in_specs=[
            pl.BlockSpec((32, R_BLK), lambda j: (0, j)),
            pl.BlockSpec((R_BLK, D), lambda j: (j, 0)),
            pl.BlockSpec((D, D), lambda j: (0, 0)),
        ],
        out_specs=[
            pl.BlockSpec((R_BLK, D), lambda j: (j, 0)),
            pl.BlockSpec((R_BLK, 1), lambda j: (j, 0)),
        ],
        out_shape=(
            jax.ShapeDtypeStruct((N_PAD, D), jnp.float32),
            jax.ShapeDtypeStruct((N_PAD, 1), jnp.float32),
        ),
    )(hist, xp, W1)


def _mid_body(acc_ref, y_ref, d_ref, b_ref, w_ref, o_ref):
    d = d_ref[...]
    h = jnp.maximum(d * (acc_ref[...] + y_ref[...]) + b_ref[...], 0.0)
    o_ref[...] = jnp.dot(h, w_ref[...], preferred_element_type=jnp.float32) * d


def _tc_mid(acc, y, dinv, b, W):
    return pl.pallas_call(
        _mid_body,
        grid=(N_PAD // R_BLK,),
        in_specs=[
            pl.BlockSpec((R_BLK, D), lambda j: (j, 0)),
            pl.BlockSpec((R_BLK, D), lambda j: (j, 0)),
            pl.BlockSpec((R_BLK, 1), lambda j: (j, 0)),
            pl.BlockSpec((1, D), lambda j: (0, 0)),
            pl.BlockSpec((D, D), lambda j: (0, 0)),
        ],
        out_specs=pl.BlockSpec((R_BLK, D), lambda j: (j, 0)),
        out_shape=jax.ShapeDtypeStruct((N_PAD, D), jnp.float32),
    )(acc, y, dinv, b.reshape(1, D), W)


def _post_body(acc_ref, y_ref, d_ref, b_ref, wv1_ref, bv1_ref, wv2_ref,
               bv2_ref, wa1_ref, ba1_ref, wa2_ref, ba2_ref, q_ref, gsum):
    j = pl.program_id(0)
    h = jnp.maximum(d_ref[...] * (acc_ref[...] + y_ref[...]) + b_ref[...], 0.0)
    rows = lax.broadcasted_iota(jnp.int32, (R_BLK, 1), 0) + j * R_BLK
    h = jnp.where(rows < N, h, 0.0)
    part = jnp.sum(h, axis=0, keepdims=True)

    @pl.when(j == 0)
    def _init():
        gsum[...] = part
        q_ref[...] = jnp.zeros((1, D), jnp.float32)

    @pl.when(j > 0)
    def _acc():
        gsum[...] = gsum[...] + part

    @pl.when(j == N_PAD // R_BLK - 1)
    def _final():
        g = gsum[...] * (1.0 / N)
        hv = jnp.maximum(
            jnp.dot(g, wv1_ref[...], preferred_element_type=jnp.float32)
            + bv1_ref[...], 0.0)
        v = jnp.dot(hv, wv2_ref[...], preferred_element_type=jnp.float32) \
            + bv2_ref[...]
        ha = jnp.maximum(
            jnp.dot(g, wa1_ref[...], preferred_element_type=jnp.float32)
            + ba1_ref[...], 0.0)
        a = jnp.dot(ha, wa2_ref[...], preferred_element_type=jnp.float32) \
            + ba2_ref[...]
        cols = lax.broadcasted_iota(jnp.int32, (1, D), 1)
        amean = jnp.sum(jnp.where(cols < 6, a, 0.0)) * (1.0 / 6.0)
        q_ref[...] = v[:, 0:1] + a - amean


def _tc_post(acc, y, dinv, b3, Wv1, bv1, Wv2p, bv2p, Wa1, ba1, Wa2p, ba2p):
    whole = pl.BlockSpec((1, D), lambda j: (0, 0))
    mat = pl.BlockSpec((D, D), lambda j: (0, 0))
    return pl.pallas_call(
        _post_body,
        grid=(N_PAD // R_BLK,),
        in_specs=[
            pl.BlockSpec((R_BLK, D), lambda j: (j, 0)),
            pl.BlockSpec((R_BLK, D), lambda j: (j, 0)),
            pl.BlockSpec((R_BLK, 1), lambda j: (j, 0)),
            whole, mat, whole, mat, whole, mat, whole, mat, whole,
        ],
        out_specs=pl.BlockSpec((1, D), lambda j: (0, 0)),
        out_shape=jax.ShapeDtypeStruct((1, D), jnp.float32),
        scratch_shapes=[pltpu.VMEM((1, D), jnp.float32)],
    )(acc, y, dinv, b3.reshape(1, D), Wv1, bv1.reshape(1, D), Wv2p,
      bv2p, Wa1, ba1.reshape(1, D), Wa2p, ba2p)


def kernel(x, edge_index, W1, b1, W2, b2, W3, b3,
           Wv1, bv1, Wv2, bv2, Wa1, ba1, Wa2, ba2):
    src = edge_index[0].astype(jnp.int32)
    dst = edge_index[1].astype(jnp.int32)
    n_extra = E_PAD - E
    pad_src = (jnp.arange(n_extra, dtype=jnp.int32) * 37) % 1024
    pad_dst = jnp.full((n_extra,), N_PAD - 1, jnp.int32)
    srcp = jnp.concatenate([src, pad_src])
    dstp = jnp.concatenate([dst, pad_dst])
    xp = jnp.pad(x, ((0, N_PAD - N), (0, 0)))

    bv2p = jnp.pad(bv2, (0, D - 1)).reshape(1, D)
    Wv2p = jnp.pad(Wv2, ((0, 0), (0, D - 1)))
    ba2p = jnp.pad(ba2, (0, D - 6)).reshape(1, D)
    Wa2p = jnp.pad(Wa2, ((0, 0), (0, D - 6)))

    hist = _sc_deg(dstp)
    gbin, sbin, cnt = _sc_bin(srcp, dstp)
    y1, dinv = _tc_k1(hist, xp, W1)
    acc1 = _sc_apply(y1, gbin, sbin, cnt)
    y2 = _tc_mid(acc1, y1, dinv, b1, W2)
    acc2 = _sc_apply(y2, gbin, sbin, cnt)
    y3 = _tc_mid(acc2, y2, dinv, b2, W3)
    acc3 = _sc_apply(y3, gbin, sbin, cnt)
    q = _tc_post(acc3, y3, dinv, b3, Wv1, bv1, Wv2p, bv2p, Wa1, ba1,
                 Wa2p, ba2p)
    return q[:, :6]
